# baseline (speedup 1.0000x reference)
import functools

import jax
import jax.numpy as jnp
from jax import lax
from jax.experimental import pallas as pl
from jax.experimental.pallas import tpu as pltpu
from jax.experimental.pallas import tpu_sc as plsc

F32 = jnp.float32
EPS = 1e-5
DIMS_NN = (((1,), (0,)), ((), ()))
DIMS_TN = (((0,), (0,)), ((), ()))


def _mm_hi(a, b):
    return lax.dot_general(a.astype(jnp.bfloat16), b.astype(jnp.bfloat16),
                           DIMS_NN, preferred_element_type=F32)


def _mm_tn(a, b):
    return lax.dot_general(a.astype(jnp.bfloat16), b.astype(jnp.bfloat16),
                           DIMS_TN, preferred_element_type=F32)


def _mm_exact(a, b):
    return lax.dot_general(a, b, DIMS_NN, preferred_element_type=F32,
                           precision=lax.Precision.HIGHEST)


def _enc_body(xp_ref, w_ref, b_ref, out_ref, st_ref):
    i = pl.program_id(0)

    @pl.when(i == 0)
    def _():
        st_ref[...] = jnp.zeros_like(st_ref)

    r0 = (i % 14) * 16
    cols = []
    for ky in range(3):
        for kx in range(3):
            for c in range(3):
                rows = [xp_ref[0, c, r0 + rr + ky, pl.ds(kx, 224)]
                        for rr in range(16)]
                cols.append(jnp.concatenate(rows, 0))
    patches = jnp.stack(cols, 0)
    y = _mm_tn(patches, w_ref[...])
    y = y + b_ref[0, :][None, :]
    out_ref[...] = y
    st_ref[0, :96] += jnp.sum(y, axis=0)
    st_ref[1, :96] += jnp.sum(y * y, axis=0)


def _encoder(x_pad, w27, b1):
    return pl.pallas_call(
        _enc_body,
        grid=(56,),
        in_specs=[
            pl.BlockSpec((1, 3, 226, 226), lambda i: (i // 14, 0, 0, 0)),
            pl.BlockSpec((27, 96), lambda i: (0, 0)),
            pl.BlockSpec((1, 96), lambda i: (0, 0)),
        ],
        out_specs=[
            pl.BlockSpec((3584, 96), lambda i: (i, 0)),
            pl.BlockSpec((8, 128), lambda i: (0, 0)),
        ],
        out_shape=[
            jax.ShapeDtypeStruct((200704, 96), F32),
            jax.ShapeDtypeStruct((8, 128), F32),
        ],
    )(x_pad, w27, b1)


def _vq_body(ze_ref, cb_ref, bnp_ref, z_ref, idx_ref, cnorm_ref):
    j = pl.program_id(0)

    @pl.when(j == 0)
    def _():
        for t in range(16):
            sl = cb_ref[pl.ds(512 * t, 512), :]
            cnorm_ref[0, pl.ds(512 * t, 512)] = jnp.sum(sl * sl, axis=1)

    g = bnp_ref[0, :96]
    bb = bnp_ref[1, :96]
    m = bnp_ref[2, :96]
    v = bnp_ref[3, :96]

    rr = lax.broadcasted_iota(jnp.int32, (112, 224), 0)
    cc = lax.broadcasted_iota(jnp.int32, (112, 224), 1)
    sel_e = (cc == 2 * rr).astype(F32)
    sel_o = (cc == 2 * rr + 1).astype(F32)

    parts = []
    for k in range(4):
        ra = ze_ref[pl.ds(448 * k, 224), :]
        rb = ze_ref[pl.ds(448 * k + 224, 224), :]
        mx = jnp.maximum(ra, rb)
        ev = _mm_exact(sel_e, mx)
        od = _mm_exact(sel_o, mx)
        parts.append(jnp.maximum(ev, od))
    pooled = jnp.concatenate(parts, 0)

    z = ((pooled - m[None, :]) / jnp.sqrt(v[None, :] + EPS)) * g[None, :] \
        + bb[None, :]
    z = jnp.maximum(z, 0.0)
    z_ref[...] = z

    lane = lax.broadcasted_iota(jnp.int32, (448, 512), 1)
    best = jnp.full((448,), jnp.inf, F32)
    bidx = jnp.zeros((448,), jnp.int32)
    for ci in range(16):
        cbc = cb_ref[pl.ds(512 * ci, 512), :]
        mm = lax.dot_general(z.astype(jnp.bfloat16),
                             cbc.astype(jnp.bfloat16),
                             (((1,), (1,)), ((), ())),
                             preferred_element_type=F32)
        d = cnorm_ref[0, pl.ds(512 * ci, 512)][None, :] - 2.0 * mm
        cmin = jnp.min(d, axis=1)
        am = jnp.min(jnp.where(d == cmin[:, None], lane, 512), axis=1) \
            + 512 * ci
        upd = cmin < best
        best = jnp.where(upd, cmin, best)
        bidx = jnp.where(upd, am, bidx)
    idx_ref[0, 0, :] = bidx


def _vq(z_enc, cb, bnp):
    return pl.pallas_call(
        _vq_body,
        grid=(112,),
        in_specs=[
            pl.BlockSpec((1792, 96), lambda j: (j, 0)),
            pl.BlockSpec((8192, 96), lambda j: (0, 0)),
            pl.BlockSpec((8, 128), lambda j: (0, 0)),
        ],
        out_specs=[
            pl.BlockSpec((448, 96), lambda j: (j, 0)),
            pl.BlockSpec((1, 1, 448), lambda j: (j, 0, 0)),
        ],
        out_shape=[
            jax.ShapeDtypeStruct((50176, 96), F32),
            jax.ShapeDtypeStruct((112, 1, 448), jnp.int32),
        ],
        scratch_shapes=[pltpu.VMEM((1, 8192), F32)],
    )(z_enc, cb, bnp)


def _sc_gather(codebook, idx_flat):
    info = plsc.get_sparse_core_info()
    nw = info.num_cores * info.num_subcores
    b_tot, d = idx_flat.shape[0], codebook.shape[1]
    b_per_w = b_tot // nw
    ch = 112
    n_ch = b_per_w // ch
    mesh = plsc.VectorSubcoreMesh(core_axis_name="c", subcore_axis_name="s")

    @functools.partial(
        pl.kernel, mesh=mesh,
        out_type=jax.ShapeDtypeStruct((b_tot, d), F32),
        scratch_types=[
            pltpu.VMEM((ch,), jnp.int32),
            pltpu.VMEM((ch, d), F32),
            pltpu.SemaphoreType.DMA,
        ],
    )
    def gk(cb_hbm, idx_hbm, out_hbm, idx_v, rows_v, sem):
        wid = lax.axis_index("s") * info.num_cores + lax.axis_index("c")
        base = wid * b_per_w
        for c in range(n_ch):
            off = base + c * ch
            pltpu.sync_copy(idx_hbm.at[pl.ds(off, ch)], idx_v)
            pltpu.async_copy(cb_hbm.at[idx_v], rows_v, sem).wait()
            pltpu.sync_copy(rows_v, out_hbm.at[pl.ds(off, ch)])

    return gk(codebook, idx_flat)


def _ups_body(z_ref, q_ref, u_ref, acc_ref):
    s = pl.program_id(0)

    @pl.when(s == 0)
    def _():
        acc_ref[...] = jnp.zeros_like(acc_ref)

    zb = z_ref[...]
    qb = q_ref[:, :96]
    diff = qb - zb
    hq = zb + diff
    acc_ref[0, :96] += jnp.sum(diff * diff, axis=0)

    xr = lax.broadcasted_iota(jnp.int32, (224, 112), 0)
    hc = lax.broadcasted_iota(jnp.int32, (224, 112), 1)
    dup = (hc == xr // 2).astype(F32)

    u_ref[...] = jnp.zeros_like(u_ref)
    for i in range(8):
        row = hq[112 * i:112 * i + 112, :]
        up = _mm_exact(dup, row)
        u_ref[2 * i, pl.ds(1, 224), :] = up
        u_ref[2 * i + 1, pl.ds(1, 224), :] = up


def _upsample(z, q):
    return pl.pallas_call(
        _ups_body,
        grid=(56,),
        in_specs=[
            pl.BlockSpec((896, 96), lambda s: (s, 0)),
            pl.BlockSpec((896, 128), lambda s: (s, 0)),
        ],
        out_specs=[
            pl.BlockSpec((16, 226, 96), lambda s: (s, 0, 0)),
            pl.BlockSpec((8, 128), lambda s: (0, 0)),
        ],
        out_shape=[
            jax.ShapeDtypeStruct((896, 226, 96), F32),
            jax.ShapeDtypeStruct((8, 128), F32),
        ],
    )(z, q)


def _dec_mm_body(u_ref, w_ref, t_ref):
    w = w_ref[...]
    for i in range(16):
        t_ref[i] = _mm_hi(u_ref[i], w)


def _dec_mm(u, w2r):
    return pl.pallas_call(
        _dec_mm_body,
        grid=(56,),
        in_specs=[
            pl.BlockSpec((16, 226, 96), lambda s: (s, 0, 0)),
            pl.BlockSpec((96, 27), lambda s: (0, 0)),
        ],
        out_specs=pl.BlockSpec((16, 226, 27), lambda s: (s, 0, 0)),
        out_shape=jax.ShapeDtypeStruct((896, 226, 27), F32),
    )(u, w2r)


def _dec_sh_body(tc_ref, tp_ref, tn_ref, b2_ref, y_ref, st_ref):
    s = pl.program_id(0)
    jj = s % 14

    @pl.when(s == 0)
    def _():
        st_ref[...] = jnp.zeros_like(st_ref)

    prev_row = jnp.where(jj == 0, 0.0, tp_ref[15])
    next_row = jnp.where(jj == 13, 0.0, tn_ref[0])
    win = jnp.concatenate([prev_row[None], tc_ref[...], next_row[None]], 0)
    wins = [win[:, kx:kx + 224, :] for kx in range(3)]

    b2 = b2_ref[0, :3]
    ssum = jnp.zeros((3,), F32)
    ssq = jnp.zeros((3,), F32)
    for i in range(16):
        accv = jnp.broadcast_to(b2[None, :], (224, 3))
        for ky in range(3):
            for kx in range(3):
                col = 3 * (3 * ky + kx)
                accv = accv + wins[kx][i + ky][:, col:col + 3]
        y_ref[i] = accv
        ssum = ssum + jnp.sum(accv, axis=0)
        ssq = ssq + jnp.sum(accv * accv, axis=0)
    st_ref[0, :3] += ssum
    st_ref[1, :3] += ssq


def _dec_sh(t, b2):
    def pmap(s):
        return (jnp.where(s % 14 == 0, s, s - 1), 0, 0)

    def nmap(s):
        return (jnp.where(s % 14 == 13, s, s + 1), 0, 0)

    return pl.pallas_call(
        _dec_sh_body,
        grid=(56,),
        in_specs=[
            pl.BlockSpec((16, 226, 27), lambda s: (s, 0, 0)),
            pl.BlockSpec((16, 226, 27), pmap),
            pl.BlockSpec((16, 226, 27), nmap),
            pl.BlockSpec((1, 128), lambda s: (0, 0)),
        ],
        out_specs=[
            pl.BlockSpec((16, 224, 3), lambda s: (s, 0, 0)),
            pl.BlockSpec((8, 128), lambda s: (0, 0)),
        ],
        out_shape=[
            jax.ShapeDtypeStruct((896, 224, 3), F32),
            jax.ShapeDtypeStruct((8, 128), F32),
        ],
    )(t, t, t, b2)


def _bn2_body(y_ref, bnp_ref, out_ref):
    g = bnp_ref[0, :3]
    bb = bnp_ref[1, :3]
    m = bnp_ref[2, :3]
    v = bnp_ref[3, :3]
    yv = y_ref[...]
    xh = (yv - m[None, None, :]) / jnp.sqrt(v[None, None, :] + EPS)
    out_ref[...] = jnp.tanh(xh * g[None, None, :] + bb[None, None, :])


def _bn2(y_pre, bnp2):
    return pl.pallas_call(
        _bn2_body,
        grid=(56,),
        in_specs=[
            pl.BlockSpec((16, 224, 3), lambda s: (s, 0, 0)),
            pl.BlockSpec((8, 128), lambda s: (0, 0)),
        ],
        out_specs=pl.BlockSpec((16, 224, 3), lambda s: (s, 0, 0)),
        out_shape=jax.ShapeDtypeStruct((896, 224, 3), F32),
    )(y_pre, bnp2)


def kernel(x, conv1_w, conv1_b, bn1_g, bn1_b, codebook,
           conv2_w, conv2_b, bn2_g, bn2_b):
    n1 = jnp.float32(4 * 224 * 224)

    x_pad = jnp.pad(x, ((0, 0), (0, 0), (1, 1), (1, 1)))
    w27 = jnp.transpose(conv1_w, (2, 3, 1, 0)).reshape(27, 96)
    z_enc, st1 = _encoder(x_pad, w27, conv1_b.reshape(1, 96))

    mean1 = st1[0, :96] / n1
    var1 = st1[1, :96] / n1 - mean1 * mean1
    bnp1 = jnp.zeros((8, 128), F32)
    bnp1 = bnp1.at[0, :96].set(bn1_g).at[1, :96].set(bn1_b)
    bnp1 = bnp1.at[2, :96].set(mean1).at[3, :96].set(var1)

    z, idx3 = _vq(z_enc, codebook, bnp1)
    idx_flat = idx3.reshape(50176)

    cb_pad = jnp.pad(codebook, ((0, 0), (0, 32)))
    q = _sc_gather(cb_pad, idx_flat)

    u, acc = _upsample(z, q)
    commit_loss = jnp.float32(0.25) * (jnp.sum(acc[0, :96])
                                       / jnp.float32(50176 * 96))

    w2r = jnp.transpose(conv2_w, (1, 2, 3, 0)).reshape(96, 27)
    t = _dec_mm(u, w2r)
    y_pre, st2 = _dec_sh(t, jnp.pad(conv2_b, (0, 125)).reshape(1, 128))

    mean2 = st2[0, :3] / n1
    var2 = st2[1, :3] / n1 - mean2 * mean2
    bnp2 = jnp.zeros((8, 128), F32)
    bnp2 = bnp2.at[0, :3].set(bn2_g).at[1, :3].set(bn2_b)
    bnp2 = bnp2.at[2, :3].set(mean2).at[3, :3].set(var2)

    y_chl = _bn2(y_pre, bnp2)
    y = jnp.transpose(y_chl.reshape(4, 224, 224, 3), (0, 3, 1, 2))

    return y, idx_flat.reshape(4, 112, 112), commit_loss


# speedup vs baseline: 1.3227x; 1.3227x over previous
import functools

import jax
import jax.numpy as jnp
from jax import lax
from jax.experimental import pallas as pl
from jax.experimental.pallas import tpu as pltpu
from jax.experimental.pallas import tpu_sc as plsc

F32 = jnp.float32
EPS = 1e-5
DIMS_NN = (((1,), (0,)), ((), ()))
DIMS_TN = (((0,), (0,)), ((), ()))


def _mm_hi(a, b):
    return lax.dot_general(a.astype(jnp.bfloat16), b.astype(jnp.bfloat16),
                           DIMS_NN, preferred_element_type=F32)


def _mm_tn(a, b):
    return lax.dot_general(a.astype(jnp.bfloat16), b.astype(jnp.bfloat16),
                           DIMS_TN, preferred_element_type=F32)


def _mm_exact(a, b):
    return lax.dot_general(a, b, DIMS_NN, preferred_element_type=F32,
                           precision=lax.Precision.HIGHEST)


def _mm_exact_tn(a, b):
    return lax.dot_general(a, b, DIMS_TN, preferred_element_type=F32,
                           precision=lax.Precision.HIGHEST)


def _enc_body(xp_ref, w_ref, b_ref, out_ref, st_ref):
    i = pl.program_id(0)

    @pl.when(i == 0)
    def _():
        st_ref[...] = jnp.zeros_like(st_ref)

    r0 = (i % 14) * 16
    cols = []
    for ky in range(3):
        for kx in range(3):
            for c in range(3):
                rows = [xp_ref[0, c, r0 + rr + ky, pl.ds(kx, 224)]
                        for rr in range(16)]
                cols.append(jnp.concatenate(rows, 0))
    patches = jnp.stack(cols, 0)
    y = _mm_tn(patches, w_ref[...])
    y = y + b_ref[0, :][None, :]
    out_ref[...] = y
    st_ref[0, :96] += jnp.sum(y, axis=0)
    st_ref[1, :96] += jnp.sum(y * y, axis=0)


def _encoder(x_pad, w27, b1):
    return pl.pallas_call(
        _enc_body,
        grid=(56,),
        in_specs=[
            pl.BlockSpec((1, 3, 226, 226), lambda i: (i // 14, 0, 0, 0)),
            pl.BlockSpec((27, 96), lambda i: (0, 0)),
            pl.BlockSpec((1, 96), lambda i: (0, 0)),
        ],
        out_specs=[
            pl.BlockSpec((3584, 96), lambda i: (i, 0)),
            pl.BlockSpec((8, 128), lambda i: (0, 0)),
        ],
        out_shape=[
            jax.ShapeDtypeStruct((200704, 96), F32),
            jax.ShapeDtypeStruct((8, 128), F32),
        ],
    )(x_pad, w27, b1)


def _vq_body(ze_ref, cb_ref, bnp_ref, z_ref, idx_ref, cnorm_ref):
    j = pl.program_id(0)

    @pl.when(j == 0)
    def _():
        for t in range(16):
            sl = cb_ref[pl.ds(512 * t, 512), :]
            cnorm_ref[0, pl.ds(512 * t, 512)] = jnp.sum(sl * sl, axis=1)

    g = bnp_ref[0, :96]
    bb = bnp_ref[1, :96]
    m = bnp_ref[2, :96]
    v = bnp_ref[3, :96]

    rr = lax.broadcasted_iota(jnp.int32, (112, 224), 0)
    cc = lax.broadcasted_iota(jnp.int32, (112, 224), 1)
    sel_e = (cc == 2 * rr).astype(F32)
    sel_o = (cc == 2 * rr + 1).astype(F32)

    parts = []
    for k in range(4):
        ra = ze_ref[pl.ds(448 * k, 224), :]
        rb = ze_ref[pl.ds(448 * k + 224, 224), :]
        mx = jnp.maximum(ra, rb)
        ev = _mm_exact(sel_e, mx)
        od = _mm_exact(sel_o, mx)
        parts.append(jnp.maximum(ev, od))
    pooled = jnp.concatenate(parts, 0)

    z = ((pooled - m[None, :]) / jnp.sqrt(v[None, :] + EPS)) * g[None, :] \
        + bb[None, :]
    z = jnp.maximum(z, 0.0)
    z_ref[...] = z

    lane = lax.broadcasted_iota(jnp.int32, (448, 512), 1)
    best = jnp.full((448,), jnp.inf, F32)
    bidx = jnp.zeros((448,), jnp.int32)
    for ci in range(16):
        cbc = cb_ref[pl.ds(512 * ci, 512), :]
        mm = lax.dot_general(z.astype(jnp.bfloat16),
                             cbc.astype(jnp.bfloat16),
                             (((1,), (1,)), ((), ())),
                             preferred_element_type=F32)
        d = cnorm_ref[0, pl.ds(512 * ci, 512)][None, :] - 2.0 * mm
        cmin = jnp.min(d, axis=1)
        am = jnp.min(jnp.where(d == cmin[:, None], lane, 512), axis=1) \
            + 512 * ci
        upd = cmin < best
        best = jnp.where(upd, cmin, best)
        bidx = jnp.where(upd, am, bidx)
    idx_ref[0, 0, :] = bidx


def _vq(z_enc, cb, bnp):
    return pl.pallas_call(
        _vq_body,
        grid=(112,),
        in_specs=[
            pl.BlockSpec((1792, 96), lambda j: (j, 0)),
            pl.BlockSpec((8192, 96), lambda j: (0, 0)),
            pl.BlockSpec((8, 128), lambda j: (0, 0)),
        ],
        out_specs=[
            pl.BlockSpec((448, 96), lambda j: (j, 0)),
            pl.BlockSpec((1, 1, 448), lambda j: (j, 0, 0)),
        ],
        out_shape=[
            jax.ShapeDtypeStruct((50176, 96), F32),
            jax.ShapeDtypeStruct((112, 1, 448), jnp.int32),
        ],
        scratch_shapes=[pltpu.VMEM((1, 8192), F32)],
    )(z_enc, cb, bnp)


def _sc_gather(codebook, idx_flat):
    info = plsc.get_sparse_core_info()
    nw = info.num_cores * info.num_subcores
    b_tot, d = idx_flat.shape[0], codebook.shape[1]
    b_per_w = b_tot // nw
    ch = 112
    n_ch = b_per_w // ch
    mesh = plsc.VectorSubcoreMesh(core_axis_name="c", subcore_axis_name="s")

    @functools.partial(
        pl.kernel, mesh=mesh,
        out_type=jax.ShapeDtypeStruct((b_tot, d), F32),
        scratch_types=[
            pltpu.VMEM((ch,), jnp.int32),
            pltpu.VMEM((ch, d), F32),
            pltpu.SemaphoreType.DMA,
        ],
    )
    def gk(cb_hbm, idx_hbm, out_hbm, idx_v, rows_v, sem):
        wid = lax.axis_index("s") * info.num_cores + lax.axis_index("c")
        base = wid * b_per_w
        for c in range(n_ch):
            off = base + c * ch
            pltpu.sync_copy(idx_hbm.at[pl.ds(off, ch)], idx_v)
            pltpu.async_copy(cb_hbm.at[idx_v], rows_v, sem).wait()
            pltpu.sync_copy(rows_v, out_hbm.at[pl.ds(off, ch)])

    return gk(codebook, idx_flat)


def _dec_body(zc_ref, qc_ref, zp_ref, qp_ref, zn_ref, qn_ref,
              w_ref, b2_ref, y_ref, st_ref):
    s = pl.program_id(0)
    jj = s % 14

    @pl.when(s == 0)
    def _():
        st_ref[...] = jnp.zeros_like(st_ref)

    zc = zc_ref[...]
    qc = qc_ref[:, :96]
    diff = qc - zc
    hq_c = zc + diff
    st_ref[2, :96] += jnp.sum(diff * diff, axis=0)

    hq_p = jnp.where(jj == 0, 0.0,
                     zp_ref[...] + (qp_ref[:, :96] - zp_ref[...]))
    hq_n = jnp.where(jj == 13, 0.0,
                     zn_ref[...] + (qn_ref[:, :96] - zn_ref[...]))
    hq = jnp.concatenate([hq_p, hq_c, hq_n], 0)

    hh = lax.broadcasted_iota(jnp.int32, (112, 226), 0)
    xx = lax.broadcasted_iota(jnp.int32, (112, 226), 1)
    dup_t = ((xx >= 1) & (xx <= 224) & (hh == (xx - 1) // 2)).astype(F32)

    w = w_ref[...]
    t_rows = []
    for hr in range(10):
        hw = _mm_hi(hq[112 * hr:112 * hr + 112, :], w)
        t_rows.append(_mm_exact_tn(hw, dup_t))

    b2 = b2_ref[0, :3]
    ssum = jnp.zeros((3,), F32)
    ssq = jnp.zeros((3,), F32)
    for i in range(16):
        acc = jnp.broadcast_to(b2[:, None], (3, 224))
        for ky in range(3):
            hloc = (i + ky - 1) // 2 + 1
            trow = t_rows[hloc]
            for kx in range(3):
                col = 3 * (3 * ky + kx)
                acc = acc + trow[col:col + 3, kx:kx + 224]
        y_ref[:, i, :] = acc
        ssum = ssum + jnp.sum(acc, axis=1)
        ssq = ssq + jnp.sum(acc * acc, axis=1)
    st_ref[0, :3] += ssum
    st_ref[1, :3] += ssq


def _decoder(z, q, w2r, b2):
    def hmap_prev(s):
        return (jnp.where(s % 14 == 0, 0, 8 * s - 1), 0)

    def hmap_next(s):
        return (jnp.where(s % 14 == 13, 0, 8 * s + 8), 0)

    return pl.pallas_call(
        _dec_body,
        grid=(56,),
        in_specs=[
            pl.BlockSpec((896, 96), lambda s: (s, 0)),
            pl.BlockSpec((896, 128), lambda s: (s, 0)),
            pl.BlockSpec((112, 96), hmap_prev),
            pl.BlockSpec((112, 128), hmap_prev),
            pl.BlockSpec((112, 96), hmap_next),
            pl.BlockSpec((112, 128), hmap_next),
            pl.BlockSpec((96, 27), lambda s: (0, 0)),
            pl.BlockSpec((1, 128), lambda s: (0, 0)),
        ],
        out_specs=[
            pl.BlockSpec((3, 16, 224), lambda s: (0, s, 0)),
            pl.BlockSpec((8, 128), lambda s: (0, 0)),
        ],
        out_shape=[
            jax.ShapeDtypeStruct((3, 896, 224), F32),
            jax.ShapeDtypeStruct((8, 128), F32),
        ],
    )(z, q, z, q, z, q, w2r, b2)


def _bn2_body(y_ref, bnp_ref, out_ref):
    g = bnp_ref[0, :3]
    bb = bnp_ref[1, :3]
    m = bnp_ref[2, :3]
    v = bnp_ref[3, :3]
    yv = y_ref[...]
    xh = (yv - m[:, None, None]) / jnp.sqrt(v[:, None, None] + EPS)
    out_ref[...] = jnp.tanh(xh * g[:, None, None] + bb[:, None, None])


def _bn2(y_pre, bnp2):
    return pl.pallas_call(
        _bn2_body,
        grid=(8,),
        in_specs=[
            pl.BlockSpec((3, 112, 224), lambda s: (0, s, 0)),
            pl.BlockSpec((8, 128), lambda s: (0, 0)),
        ],
        out_specs=pl.BlockSpec((3, 112, 224), lambda s: (0, s, 0)),
        out_shape=jax.ShapeDtypeStruct((3, 896, 224), F32),
    )(y_pre, bnp2)


def kernel(x, conv1_w, conv1_b, bn1_g, bn1_b, codebook,
           conv2_w, conv2_b, bn2_g, bn2_b):
    n1 = jnp.float32(4 * 224 * 224)

    x_pad = jnp.pad(x, ((0, 0), (0, 0), (1, 1), (1, 1)))
    w27 = jnp.transpose(conv1_w, (2, 3, 1, 0)).reshape(27, 96)
    z_enc, st1 = _encoder(x_pad, w27, conv1_b.reshape(1, 96))

    mean1 = st1[0, :96] / n1
    var1 = st1[1, :96] / n1 - mean1 * mean1
    bnp1 = jnp.zeros((8, 128), F32)
    bnp1 = bnp1.at[0, :96].set(bn1_g).at[1, :96].set(bn1_b)
    bnp1 = bnp1.at[2, :96].set(mean1).at[3, :96].set(var1)

    z, idx3 = _vq(z_enc, codebook, bnp1)
    idx_flat = idx3.reshape(50176)

    cb_pad = jnp.pad(codebook, ((0, 0), (0, 32)))
    q = _sc_gather(cb_pad, idx_flat)

    w2r = jnp.transpose(conv2_w, (1, 2, 3, 0)).reshape(96, 27)
    y_pre, st2 = _decoder(z, q, w2r,
                          jnp.pad(conv2_b, (0, 125)).reshape(1, 128))
    commit_loss = jnp.float32(0.25) * (jnp.sum(st2[2, :96])
                                       / jnp.float32(50176 * 96))

    mean2 = st2[0, :3] / n1
    var2 = st2[1, :3] / n1 - mean2 * mean2
    bnp2 = jnp.zeros((8, 128), F32)
    bnp2 = bnp2.at[0, :3].set(bn2_g).at[1, :3].set(bn2_b)
    bnp2 = bnp2.at[2, :3].set(mean2).at[3, :3].set(var2)

    y_t = _bn2(y_pre, bnp2)
    y = jnp.transpose(y_t.reshape(3, 4, 224, 224), (1, 0, 2, 3))

    return y, idx_flat.reshape(4, 112, 112), commit_loss


# speedup vs baseline: 1.5125x; 1.1435x over previous
import functools

import jax
import jax.numpy as jnp
from jax import lax
from jax.experimental import pallas as pl
from jax.experimental.pallas import tpu as pltpu
from jax.experimental.pallas import tpu_sc as plsc

F32 = jnp.float32
EPS = 1e-5
DIMS_NN = (((1,), (0,)), ((), ()))
DIMS_TN = (((0,), (0,)), ((), ()))


def _mm_hi(a, b):
    return lax.dot_general(a.astype(jnp.bfloat16), b.astype(jnp.bfloat16),
                           DIMS_NN, preferred_element_type=F32)


def _mm_tn(a, b):
    return lax.dot_general(a.astype(jnp.bfloat16), b.astype(jnp.bfloat16),
                           DIMS_TN, preferred_element_type=F32)


def _mm_exact(a, b):
    return lax.dot_general(a, b, DIMS_NN, preferred_element_type=F32,
                           precision=lax.Precision.HIGHEST)


def _mm_exact_tn(a, b):
    return lax.dot_general(a, b, DIMS_TN, preferred_element_type=F32,
                           precision=lax.Precision.HIGHEST)


def _enc_body(xp_ref, w_ref, b_ref, out_ref, st_ref):
    i = pl.program_id(0)

    @pl.when(i == 0)
    def _():
        st_ref[...] = jnp.zeros_like(st_ref)

    r0 = (i % 14) * 16
    cols = []
    for ky in range(3):
        for kx in range(3):
            for c in range(3):
                rows = [xp_ref[0, c, r0 + rr + ky, pl.ds(kx, 224)]
                        for rr in range(16)]
                cols.append(jnp.concatenate(rows, 0))
    patches = jnp.stack(cols, 0)
    y = _mm_tn(patches, w_ref[...])
    y = y + b_ref[0, :][None, :]
    out_ref[...] = y
    st_ref[0, :96] += jnp.sum(y, axis=0)
    st_ref[1, :96] += jnp.sum(y * y, axis=0)


def _encoder(x_pad, w27, b1):
    return pl.pallas_call(
        _enc_body,
        grid=(56,),
        in_specs=[
            pl.BlockSpec((1, 3, 226, 226), lambda i: (i // 14, 0, 0, 0)),
            pl.BlockSpec((27, 96), lambda i: (0, 0)),
            pl.BlockSpec((1, 96), lambda i: (0, 0)),
        ],
        out_specs=[
            pl.BlockSpec((3584, 96), lambda i: (i, 0)),
            pl.BlockSpec((8, 128), lambda i: (0, 0)),
        ],
        out_shape=[
            jax.ShapeDtypeStruct((200704, 96), F32),
            jax.ShapeDtypeStruct((8, 128), F32),
        ],
    )(x_pad, w27, b1)


def _vq_body(ze_ref, cb_ref, bnp_ref, z_ref, idx_ref, cnorm_ref):
    j = pl.program_id(0)

    @pl.when(j == 0)
    def _():
        for t in range(16):
            sl = cb_ref[pl.ds(512 * t, 512), :]
            cnorm_ref[0, pl.ds(512 * t, 512)] = \
                jnp.sum(sl * sl, axis=1) * 0.5

    g = bnp_ref[0, :96]
    bb = bnp_ref[1, :96]
    m = bnp_ref[2, :96]
    v = bnp_ref[3, :96]

    rr = lax.broadcasted_iota(jnp.int32, (112, 224), 0)
    cc = lax.broadcasted_iota(jnp.int32, (112, 224), 1)
    sel_e = (cc == 2 * rr).astype(F32)
    sel_o = (cc == 2 * rr + 1).astype(F32)

    parts = []
    for k in range(4):
        ra = ze_ref[pl.ds(448 * k, 224), :]
        rb = ze_ref[pl.ds(448 * k + 224, 224), :]
        mx = jnp.maximum(ra, rb)
        ev = _mm_exact(sel_e, mx)
        od = _mm_exact(sel_o, mx)
        parts.append(jnp.maximum(ev, od))
    pooled = jnp.concatenate(parts, 0)

    z = ((pooled - m[None, :]) / jnp.sqrt(v[None, :] + EPS)) * g[None, :] \
        + bb[None, :]
    z = jnp.maximum(z, 0.0)
    z_ref[...] = z

    lane = lax.broadcasted_iota(jnp.int32, (448, 512), 1)
    macc = jnp.full((448, 512), -jnp.inf, F32)
    iacc = jnp.zeros((448, 512), jnp.int32)
    zb16 = z.astype(jnp.bfloat16)
    for ci in range(16):
        cbc = cb_ref[pl.ds(512 * ci, 512), :]
        mm = lax.dot_general(zb16, cbc.astype(jnp.bfloat16),
                             (((1,), (1,)), ((), ())),
                             preferred_element_type=F32)
        sv = mm - cnorm_ref[0, pl.ds(512 * ci, 512)][None, :]
        better = sv > macc
        macc = jnp.where(better, sv, macc)
        iacc = jnp.where(better, lane + 512 * ci, iacc)
    gmax = jnp.max(macc, axis=1)
    bidx = jnp.min(jnp.where(macc == gmax[:, None], iacc, 8192), axis=1)
    idx_ref[0, 0, :] = bidx


def _vq(z_enc, cb, bnp):
    return pl.pallas_call(
        _vq_body,
        grid=(112,),
        in_specs=[
            pl.BlockSpec((1792, 96), lambda j: (j, 0)),
            pl.BlockSpec((8192, 96), lambda j: (0, 0)),
            pl.BlockSpec((8, 128), lambda j: (0, 0)),
        ],
        out_specs=[
            pl.BlockSpec((448, 96), lambda j: (j, 0)),
            pl.BlockSpec((1, 1, 448), lambda j: (j, 0, 0)),
        ],
        out_shape=[
            jax.ShapeDtypeStruct((50176, 96), F32),
            jax.ShapeDtypeStruct((112, 1, 448), jnp.int32),
        ],
        scratch_shapes=[pltpu.VMEM((1, 8192), F32)],
    )(z_enc, cb, bnp)


def _sc_gather(codebook, idx_flat):
    info = plsc.get_sparse_core_info()
    nw = info.num_cores * info.num_subcores
    b_tot, d = idx_flat.shape[0], codebook.shape[1]
    b_per_w = b_tot // nw
    ch = 112
    n_ch = b_per_w // ch
    mesh = plsc.VectorSubcoreMesh(core_axis_name="c", subcore_axis_name="s")

    @functools.partial(
        pl.kernel, mesh=mesh,
        out_type=jax.ShapeDtypeStruct((b_tot, d), F32),
        scratch_types=[
            pltpu.VMEM((ch,), jnp.int32),
            pltpu.VMEM((ch, d), F32),
            pltpu.SemaphoreType.DMA,
        ],
    )
    def gk(cb_hbm, idx_hbm, out_hbm, idx_v, rows_v, sem):
        wid = lax.axis_index("s") * info.num_cores + lax.axis_index("c")
        base = wid * b_per_w
        for c in range(n_ch):
            off = base + c * ch
            pltpu.sync_copy(idx_hbm.at[pl.ds(off, ch)], idx_v)
            pltpu.async_copy(cb_hbm.at[idx_v], rows_v, sem).wait()
            pltpu.sync_copy(rows_v, out_hbm.at[pl.ds(off, ch)])

    return gk(codebook, idx_flat)


def _dec_body(zc_ref, qc_ref, zp_ref, qp_ref, zn_ref, qn_ref,
              w_ref, b2_ref, y_ref, st_ref):
    s = pl.program_id(0)
    jj = s % 14

    @pl.when(s == 0)
    def _():
        st_ref[...] = jnp.zeros_like(st_ref)

    zc = zc_ref[...]
    qc = qc_ref[:, :96]
    diff = qc - zc
    hq_c = zc + diff
    st_ref[2, :96] += jnp.sum(diff * diff, axis=0)

    hq_p = jnp.where(jj == 0, 0.0,
                     zp_ref[...] + (qp_ref[:, :96] - zp_ref[...]))
    hq_n = jnp.where(jj == 13, 0.0,
                     zn_ref[...] + (qn_ref[:, :96] - zn_ref[...]))
    hq = jnp.concatenate([hq_p, hq_c, hq_n], 0)

    hh = lax.broadcasted_iota(jnp.int32, (112, 226), 0)
    xx = lax.broadcasted_iota(jnp.int32, (112, 226), 1)
    dup_t = ((xx >= 1) & (xx <= 224) & (hh == (xx - 1) // 2)).astype(F32)

    w = w_ref[...]
    t_rows = []
    for hr in range(10):
        hw = _mm_hi(hq[112 * hr:112 * hr + 112, :], w)
        t_rows.append(_mm_exact_tn(hw, dup_t))

    b2 = b2_ref[0, :3]
    ssum = jnp.zeros((3,), F32)
    ssq = jnp.zeros((3,), F32)
    for i in range(16):
        acc = jnp.broadcast_to(b2[:, None], (3, 224))
        for ky in range(3):
            hloc = (i + ky - 1) // 2 + 1
            trow = t_rows[hloc]
            for kx in range(3):
                col = 3 * (3 * ky + kx)
                acc = acc + trow[col:col + 3, kx:kx + 224]
        y_ref[:, i, :] = acc
        ssum = ssum + jnp.sum(acc, axis=1)
        ssq = ssq + jnp.sum(acc * acc, axis=1)
    st_ref[0, :3] += ssum
    st_ref[1, :3] += ssq


def _decoder(z, q, w2r, b2):
    def hmap_prev(s):
        return (jnp.where(s % 14 == 0, 0, 8 * s - 1), 0)

    def hmap_next(s):
        return (jnp.where(s % 14 == 13, 0, 8 * s + 8), 0)

    return pl.pallas_call(
        _dec_body,
        grid=(56,),
        in_specs=[
            pl.BlockSpec((896, 96), lambda s: (s, 0)),
            pl.BlockSpec((896, 128), lambda s: (s, 0)),
            pl.BlockSpec((112, 96), hmap_prev),
            pl.BlockSpec((112, 128), hmap_prev),
            pl.BlockSpec((112, 96), hmap_next),
            pl.BlockSpec((112, 128), hmap_next),
            pl.BlockSpec((96, 27), lambda s: (0, 0)),
            pl.BlockSpec((1, 128), lambda s: (0, 0)),
        ],
        out_specs=[
            pl.BlockSpec((3, 16, 224), lambda s: (0, s, 0)),
            pl.BlockSpec((8, 128), lambda s: (0, 0)),
        ],
        out_shape=[
            jax.ShapeDtypeStruct((3, 896, 224), F32),
            jax.ShapeDtypeStruct((8, 128), F32),
        ],
    )(z, q, z, q, z, q, w2r, b2)


def _bn2_body(y_ref, bnp_ref, out_ref):
    g = bnp_ref[0, :3]
    bb = bnp_ref[1, :3]
    m = bnp_ref[2, :3]
    v = bnp_ref[3, :3]
    yv = y_ref[...]
    xh = (yv - m[:, None, None]) / jnp.sqrt(v[:, None, None] + EPS)
    out_ref[...] = jnp.tanh(xh * g[:, None, None] + bb[:, None, None])


def _bn2(y_pre, bnp2):
    return pl.pallas_call(
        _bn2_body,
        grid=(8,),
        in_specs=[
            pl.BlockSpec((3, 112, 224), lambda s: (0, s, 0)),
            pl.BlockSpec((8, 128), lambda s: (0, 0)),
        ],
        out_specs=pl.BlockSpec((3, 112, 224), lambda s: (0, s, 0)),
        out_shape=jax.ShapeDtypeStruct((3, 896, 224), F32),
    )(y_pre, bnp2)


def kernel(x, conv1_w, conv1_b, bn1_g, bn1_b, codebook,
           conv2_w, conv2_b, bn2_g, bn2_b):
    n1 = jnp.float32(4 * 224 * 224)

    x_pad = jnp.pad(x, ((0, 0), (0, 0), (1, 1), (1, 1)))
    w27 = jnp.transpose(conv1_w, (2, 3, 1, 0)).reshape(27, 96)
    z_enc, st1 = _encoder(x_pad, w27, conv1_b.reshape(1, 96))

    mean1 = st1[0, :96] / n1
    var1 = st1[1, :96] / n1 - mean1 * mean1
    bnp1 = jnp.zeros((8, 128), F32)
    bnp1 = bnp1.at[0, :96].set(bn1_g).at[1, :96].set(bn1_b)
    bnp1 = bnp1.at[2, :96].set(mean1).at[3, :96].set(var1)

    z, idx3 = _vq(z_enc, codebook, bnp1)
    idx_flat = idx3.reshape(50176)

    cb_pad = jnp.pad(codebook, ((0, 0), (0, 32)))
    q = _sc_gather(cb_pad, idx_flat)

    w2r = jnp.transpose(conv2_w, (1, 2, 3, 0)).reshape(96, 27)
    y_pre, st2 = _decoder(z, q, w2r,
                          jnp.pad(conv2_b, (0, 125)).reshape(1, 128))
    commit_loss = jnp.float32(0.25) * (jnp.sum(st2[2, :96])
                                       / jnp.float32(50176 * 96))

    mean2 = st2[0, :3] / n1
    var2 = st2[1, :3] / n1 - mean2 * mean2
    bnp2 = jnp.zeros((8, 128), F32)
    bnp2 = bnp2.at[0, :3].set(bn2_g).at[1, :3].set(bn2_b)
    bnp2 = bnp2.at[2, :3].set(mean2).at[3, :3].set(var2)

    y_t = _bn2(y_pre, bnp2)
    y = jnp.transpose(y_t.reshape(3, 4, 224, 224), (1, 0, 2, 3))

    return y, idx_flat.reshape(4, 112, 112), commit_loss


# speedup vs baseline: 1.5144x; 1.0012x over previous
import functools

import jax
import jax.numpy as jnp
from jax import lax
from jax.experimental import pallas as pl
from jax.experimental.pallas import tpu as pltpu
from jax.experimental.pallas import tpu_sc as plsc

F32 = jnp.float32
EPS = 1e-5
DIMS_NN = (((1,), (0,)), ((), ()))
DIMS_TN = (((0,), (0,)), ((), ()))


def _mm_hi(a, b):
    return lax.dot_general(a.astype(jnp.bfloat16), b.astype(jnp.bfloat16),
                           DIMS_NN, preferred_element_type=F32)


def _mm_tn(a, b):
    return lax.dot_general(a.astype(jnp.bfloat16), b.astype(jnp.bfloat16),
                           DIMS_TN, preferred_element_type=F32)


def _mm_exact(a, b):
    return lax.dot_general(a, b, DIMS_NN, preferred_element_type=F32,
                           precision=lax.Precision.HIGHEST)


def _mm_exact_tn(a, b):
    return lax.dot_general(a, b, DIMS_TN, preferred_element_type=F32,
                           precision=lax.Precision.HIGHEST)


def _enc_body(xp_ref, w_ref, b_ref, out_ref, st_ref):
    i = pl.program_id(0)

    @pl.when(i == 0)
    def _():
        st_ref[...] = jnp.zeros_like(st_ref)

    r0 = (i % 14) * 16
    cols = []
    for ky in range(3):
        for kx in range(3):
            for c in range(3):
                rows = [xp_ref[0, c, r0 + rr + ky, pl.ds(kx, 224)]
                        for rr in range(16)]
                cols.append(jnp.concatenate(rows, 0))
    patches = jnp.stack(cols, 0)
    y = _mm_tn(patches, w_ref[...])
    y = y + b_ref[0, :][None, :]
    out_ref[...] = y
    st_ref[0, :96] += jnp.sum(y, axis=0)
    st_ref[1, :96] += jnp.sum(y * y, axis=0)


def _encoder(x_pad, w27, b1):
    return pl.pallas_call(
        _enc_body,
        grid=(56,),
        in_specs=[
            pl.BlockSpec((1, 3, 226, 226), lambda i: (i // 14, 0, 0, 0)),
            pl.BlockSpec((27, 96), lambda i: (0, 0)),
            pl.BlockSpec((1, 96), lambda i: (0, 0)),
        ],
        out_specs=[
            pl.BlockSpec((3584, 96), lambda i: (i, 0)),
            pl.BlockSpec((8, 128), lambda i: (0, 0)),
        ],
        out_shape=[
            jax.ShapeDtypeStruct((200704, 96), F32),
            jax.ShapeDtypeStruct((8, 128), F32),
        ],
    )(x_pad, w27, b1)


def _vq_body(ze_ref, cb_ref, bnp_ref, z_ref, idx_ref, cnorm_ref):
    j = pl.program_id(0)

    @pl.when(j == 0)
    def _():
        for t in range(16):
            sl = cb_ref[pl.ds(512 * t, 512), :]
            cnorm_ref[0, pl.ds(512 * t, 512)] = \
                jnp.sum(sl * sl, axis=1) * 0.5

    g = bnp_ref[0, :96]
    bb = bnp_ref[1, :96]
    m = bnp_ref[2, :96]
    v = bnp_ref[3, :96]

    rr = lax.broadcasted_iota(jnp.int32, (112, 224), 0)
    cc = lax.broadcasted_iota(jnp.int32, (112, 224), 1)
    sel_e = (cc == 2 * rr).astype(F32)
    sel_o = (cc == 2 * rr + 1).astype(F32)

    parts = []
    for k in range(4):
        ra = ze_ref[pl.ds(448 * k, 224), :]
        rb = ze_ref[pl.ds(448 * k + 224, 224), :]
        mx = jnp.maximum(ra, rb)
        ev = _mm_exact(sel_e, mx)
        od = _mm_exact(sel_o, mx)
        parts.append(jnp.maximum(ev, od))
    pooled = jnp.concatenate(parts, 0)

    z = ((pooled - m[None, :]) / jnp.sqrt(v[None, :] + EPS)) * g[None, :] \
        + bb[None, :]
    z = jnp.maximum(z, 0.0)
    z_ref[...] = z

    lane = lax.broadcasted_iota(jnp.int32, (448, 512), 1)
    macc = jnp.full((448, 512), -jnp.inf, F32)
    iacc = jnp.zeros((448, 512), jnp.int32)
    zb16 = z.astype(jnp.bfloat16)
    for ci in range(16):
        cbc = cb_ref[pl.ds(512 * ci, 512), :]
        mm = lax.dot_general(zb16, cbc.astype(jnp.bfloat16),
                             (((1,), (1,)), ((), ())),
                             preferred_element_type=F32)
        sv = mm - cnorm_ref[0, pl.ds(512 * ci, 512)][None, :]
        better = sv > macc
        macc = jnp.where(better, sv, macc)
        iacc = jnp.where(better, lane + 512 * ci, iacc)
    gmax = jnp.max(macc, axis=1)
    bidx = jnp.min(jnp.where(macc == gmax[:, None], iacc, 8192), axis=1)
    idx_ref[0, 0, :] = bidx


def _vq(z_enc, cb, bnp):
    return pl.pallas_call(
        _vq_body,
        grid=(112,),
        in_specs=[
            pl.BlockSpec((1792, 96), lambda j: (j, 0)),
            pl.BlockSpec((8192, 96), lambda j: (0, 0)),
            pl.BlockSpec((8, 128), lambda j: (0, 0)),
        ],
        out_specs=[
            pl.BlockSpec((448, 96), lambda j: (j, 0)),
            pl.BlockSpec((1, 1, 448), lambda j: (j, 0, 0)),
        ],
        out_shape=[
            jax.ShapeDtypeStruct((50176, 96), F32),
            jax.ShapeDtypeStruct((112, 1, 448), jnp.int32),
        ],
        scratch_shapes=[pltpu.VMEM((1, 8192), F32)],
    )(z_enc, cb, bnp)


def _sc_gather(codebook, idx_flat):
    info = plsc.get_sparse_core_info()
    nw = info.num_cores * info.num_subcores
    b_tot, d = idx_flat.shape[0], codebook.shape[1]
    b_per_w = b_tot // nw
    ch = 112
    n_ch = b_per_w // ch
    mesh = plsc.VectorSubcoreMesh(core_axis_name="c", subcore_axis_name="s")

    @functools.partial(
        pl.kernel, mesh=mesh,
        out_type=jax.ShapeDtypeStruct((b_tot, d), F32),
        scratch_types=[
            pltpu.VMEM((b_per_w,), jnp.int32),
            pltpu.VMEM((ch, d), F32),
            pltpu.VMEM((ch, d), F32),
            pltpu.SemaphoreType.DMA,
            pltpu.SemaphoreType.DMA,
        ],
    )
    def gk(cb_hbm, idx_hbm, out_hbm, idx_v, rows0, rows1, sem0, sem1):
        wid = lax.axis_index("s") * info.num_cores + lax.axis_index("c")
        base = wid * b_per_w
        pltpu.sync_copy(idx_hbm.at[pl.ds(base, b_per_w)], idx_v)
        bufs = (rows0, rows1)
        sems = (sem0, sem1)
        pend = None
        for c in range(n_ch):
            cur = pltpu.async_copy(
                cb_hbm.at[idx_v.at[pl.ds(c * ch, ch)]],
                bufs[c % 2], sems[c % 2])
            if pend is not None:
                pc, pcopy = pend
                pcopy.wait()
                pltpu.sync_copy(bufs[pc % 2],
                                out_hbm.at[pl.ds(base + pc * ch, ch)])
            pend = (c, cur)
        pc, pcopy = pend
        pcopy.wait()
        pltpu.sync_copy(bufs[pc % 2], out_hbm.at[pl.ds(base + pc * ch, ch)])

    return gk(codebook, idx_flat)


def _dec_body(zc_ref, qc_ref, zp_ref, qp_ref, zn_ref, qn_ref,
              w_ref, b2_ref, y_ref, st_ref):
    s = pl.program_id(0)
    jj = s % 14

    @pl.when(s == 0)
    def _():
        st_ref[...] = jnp.zeros_like(st_ref)

    zc = zc_ref[...]
    qc = qc_ref[:, :96]
    diff = qc - zc
    hq_c = zc + diff
    st_ref[2, :96] += jnp.sum(diff * diff, axis=0)

    hq_p = jnp.where(jj == 0, 0.0,
                     zp_ref[...] + (qp_ref[:, :96] - zp_ref[...]))
    hq_n = jnp.where(jj == 13, 0.0,
                     zn_ref[...] + (qn_ref[:, :96] - zn_ref[...]))
    hq = jnp.concatenate([hq_p, hq_c, hq_n], 0)

    hh = lax.broadcasted_iota(jnp.int32, (112, 226), 0)
    xx = lax.broadcasted_iota(jnp.int32, (112, 226), 1)
    dup_t = ((xx >= 1) & (xx <= 224) & (hh == (xx - 1) // 2)).astype(F32)

    w = w_ref[...]
    t_rows = []
    for hr in range(10):
        hw = _mm_hi(hq[112 * hr:112 * hr + 112, :], w)
        t_rows.append(_mm_exact_tn(hw, dup_t))

    b2 = b2_ref[0, :3]
    ssum = jnp.zeros((3,), F32)
    ssq = jnp.zeros((3,), F32)
    for i in range(16):
        acc = jnp.broadcast_to(b2[:, None], (3, 224))
        for ky in range(3):
            hloc = (i + ky - 1) // 2 + 1
            trow = t_rows[hloc]
            for kx in range(3):
                col = 3 * (3 * ky + kx)
                acc = acc + trow[col:col + 3, kx:kx + 224]
        y_ref[:, i, :] = acc
        ssum = ssum + jnp.sum(acc, axis=1)
        ssq = ssq + jnp.sum(acc * acc, axis=1)
    st_ref[0, :3] += ssum
    st_ref[1, :3] += ssq


def _decoder(z, q, w2r, b2):
    def hmap_prev(s):
        return (jnp.where(s % 14 == 0, 0, 8 * s - 1), 0)

    def hmap_next(s):
        return (jnp.where(s % 14 == 13, 0, 8 * s + 8), 0)

    return pl.pallas_call(
        _dec_body,
        grid=(56,),
        in_specs=[
            pl.BlockSpec((896, 96), lambda s: (s, 0)),
            pl.BlockSpec((896, 128), lambda s: (s, 0)),
            pl.BlockSpec((112, 96), hmap_prev),
            pl.BlockSpec((112, 128), hmap_prev),
            pl.BlockSpec((112, 96), hmap_next),
            pl.BlockSpec((112, 128), hmap_next),
            pl.BlockSpec((96, 27), lambda s: (0, 0)),
            pl.BlockSpec((1, 128), lambda s: (0, 0)),
        ],
        out_specs=[
            pl.BlockSpec((3, 16, 224), lambda s: (0, s, 0)),
            pl.BlockSpec((8, 128), lambda s: (0, 0)),
        ],
        out_shape=[
            jax.ShapeDtypeStruct((3, 896, 224), F32),
            jax.ShapeDtypeStruct((8, 128), F32),
        ],
    )(z, q, z, q, z, q, w2r, b2)


def _bn2_body(y_ref, bnp_ref, out_ref):
    g = bnp_ref[0, :3]
    bb = bnp_ref[1, :3]
    m = bnp_ref[2, :3]
    v = bnp_ref[3, :3]
    yv = y_ref[...]
    xh = (yv - m[:, None, None]) / jnp.sqrt(v[:, None, None] + EPS)
    out_ref[...] = jnp.tanh(xh * g[:, None, None] + bb[:, None, None])


def _bn2(y_pre, bnp2):
    return pl.pallas_call(
        _bn2_body,
        grid=(8,),
        in_specs=[
            pl.BlockSpec((3, 112, 224), lambda s: (0, s, 0)),
            pl.BlockSpec((8, 128), lambda s: (0, 0)),
        ],
        out_specs=pl.BlockSpec((3, 112, 224), lambda s: (0, s, 0)),
        out_shape=jax.ShapeDtypeStruct((3, 896, 224), F32),
    )(y_pre, bnp2)


def kernel(x, conv1_w, conv1_b, bn1_g, bn1_b, codebook,
           conv2_w, conv2_b, bn2_g, bn2_b):
    n1 = jnp.float32(4 * 224 * 224)

    x_pad = jnp.pad(x, ((0, 0), (0, 0), (1, 1), (1, 1)))
    w27 = jnp.transpose(conv1_w, (2, 3, 1, 0)).reshape(27, 96)
    z_enc, st1 = _encoder(x_pad, w27, conv1_b.reshape(1, 96))

    mean1 = st1[0, :96] / n1
    var1 = st1[1, :96] / n1 - mean1 * mean1
    bnp1 = jnp.zeros((8, 128), F32)
    bnp1 = bnp1.at[0, :96].set(bn1_g).at[1, :96].set(bn1_b)
    bnp1 = bnp1.at[2, :96].set(mean1).at[3, :96].set(var1)

    z, idx3 = _vq(z_enc, codebook, bnp1)
    idx_flat = idx3.reshape(50176)

    cb_pad = jnp.pad(codebook, ((0, 0), (0, 32)))
    q = _sc_gather(cb_pad, idx_flat)

    w2r = jnp.transpose(conv2_w, (1, 2, 3, 0)).reshape(96, 27)
    y_pre, st2 = _decoder(z, q, w2r,
                          jnp.pad(conv2_b, (0, 125)).reshape(1, 128))
    commit_loss = jnp.float32(0.25) * (jnp.sum(st2[2, :96])
                                       / jnp.float32(50176 * 96))

    mean2 = st2[0, :3] / n1
    var2 = st2[1, :3] / n1 - mean2 * mean2
    bnp2 = jnp.zeros((8, 128), F32)
    bnp2 = bnp2.at[0, :3].set(bn2_g).at[1, :3].set(bn2_b)
    bnp2 = bnp2.at[2, :3].set(mean2).at[3, :3].set(var2)

    y_t = _bn2(y_pre, bnp2)
    y = jnp.transpose(y_t.reshape(3, 4, 224, 224), (1, 0, 2, 3))

    return y, idx_flat.reshape(4, 112, 112), commit_loss


# speedup vs baseline: 1.9053x; 1.2581x over previous
import functools

import jax
import jax.numpy as jnp
from jax import lax
from jax.experimental import pallas as pl
from jax.experimental.pallas import tpu as pltpu
from jax.experimental.pallas import tpu_sc as plsc

F32 = jnp.float32
EPS = 1e-5
DIMS_NN = (((1,), (0,)), ((), ()))
DIMS_TN = (((0,), (0,)), ((), ()))


def _mm_hi(a, b):
    return lax.dot_general(a.astype(jnp.bfloat16), b.astype(jnp.bfloat16),
                           DIMS_NN, preferred_element_type=F32)


def _mm_tn(a, b):
    return lax.dot_general(a.astype(jnp.bfloat16), b.astype(jnp.bfloat16),
                           DIMS_TN, preferred_element_type=F32)


def _mm_exact(a, b):
    return lax.dot_general(a, b, DIMS_NN, preferred_element_type=F32,
                           precision=lax.Precision.HIGHEST)


def _mm_exact_tn(a, b):
    return lax.dot_general(a, b, DIMS_TN, preferred_element_type=F32,
                           precision=lax.Precision.HIGHEST)


def _enc_body(xp_ref, w_ref, b_ref, out_ref, st_ref):
    i = pl.program_id(0)

    @pl.when(i == 0)
    def _():
        st_ref[...] = jnp.zeros_like(st_ref)

    r0 = (i % 14) * 16
    cols = []
    for ky in range(3):
        for kx in range(3):
            for c in range(3):
                rows = [xp_ref[0, c, r0 + rr + ky, pl.ds(kx, 224)]
                        for rr in range(16)]
                cols.append(jnp.concatenate(rows, 0))
    patches = jnp.stack(cols, 0)
    y = _mm_tn(patches, w_ref[...])
    y = y + b_ref[0, :][None, :]
    out_ref[...] = y
    st_ref[0, :96] += jnp.sum(y, axis=0)
    st_ref[1, :96] += jnp.sum(y * y, axis=0)


def _encoder(x_pad, w27, b1):
    return pl.pallas_call(
        _enc_body,
        grid=(56,),
        in_specs=[
            pl.BlockSpec((1, 3, 226, 226), lambda i: (i // 14, 0, 0, 0)),
            pl.BlockSpec((27, 96), lambda i: (0, 0)),
            pl.BlockSpec((1, 96), lambda i: (0, 0)),
        ],
        out_specs=[
            pl.BlockSpec((3584, 96), lambda i: (i, 0)),
            pl.BlockSpec((8, 128), lambda i: (0, 0)),
        ],
        out_shape=[
            jax.ShapeDtypeStruct((200704, 96), F32),
            jax.ShapeDtypeStruct((8, 128), F32),
        ],
    )(x_pad, w27, b1)


def _vq_body(ze_ref, cb_ref, bnp_ref, z_ref, idx_ref, cnorm_ref):
    j = pl.program_id(0)

    @pl.when(j == 0)
    def _():
        for t in range(16):
            sl = cb_ref[pl.ds(512 * t, 512), :]
            cnorm_ref[0, pl.ds(512 * t, 512)] = \
                jnp.sum(sl * sl, axis=1) * 0.5

    g = bnp_ref[0, :96]
    bb = bnp_ref[1, :96]
    m = bnp_ref[2, :96]
    v = bnp_ref[3, :96]

    rr = lax.broadcasted_iota(jnp.int32, (112, 224), 0)
    cc = lax.broadcasted_iota(jnp.int32, (112, 224), 1)
    sel_e = (cc == 2 * rr).astype(F32)
    sel_o = (cc == 2 * rr + 1).astype(F32)

    parts = []
    for k in range(4):
        ra = ze_ref[pl.ds(448 * k, 224), :]
        rb = ze_ref[pl.ds(448 * k + 224, 224), :]
        mx = jnp.maximum(ra, rb)
        ev = _mm_exact(sel_e, mx)
        od = _mm_exact(sel_o, mx)
        parts.append(jnp.maximum(ev, od))
    pooled = jnp.concatenate(parts, 0)

    z = ((pooled - m[None, :]) / jnp.sqrt(v[None, :] + EPS)) * g[None, :] \
        + bb[None, :]
    z = jnp.maximum(z, 0.0)
    z_ref[...] = z

    lane = lax.broadcasted_iota(jnp.int32, (448, 512), 1)
    macc = jnp.full((448, 512), -jnp.inf, F32)
    iacc = jnp.zeros((448, 512), jnp.int32)
    zb16 = z.astype(jnp.bfloat16)
    for ci in range(16):
        cbc = cb_ref[pl.ds(512 * ci, 512), :]
        mm = lax.dot_general(zb16, cbc.astype(jnp.bfloat16),
                             (((1,), (1,)), ((), ())),
                             preferred_element_type=F32)
        sv = mm - cnorm_ref[0, pl.ds(512 * ci, 512)][None, :]
        better = sv > macc
        macc = jnp.where(better, sv, macc)
        iacc = jnp.where(better, lane + 512 * ci, iacc)
    gmax = jnp.max(macc, axis=1)
    bidx = jnp.min(jnp.where(macc == gmax[:, None], iacc, 8192), axis=1)
    idx_ref[0, 0, :] = bidx


def _vq(z_enc, cb, bnp):
    return pl.pallas_call(
        _vq_body,
        grid=(112,),
        in_specs=[
            pl.BlockSpec((1792, 96), lambda j: (j, 0)),
            pl.BlockSpec((8192, 96), lambda j: (0, 0)),
            pl.BlockSpec((8, 128), lambda j: (0, 0)),
        ],
        out_specs=[
            pl.BlockSpec((448, 96), lambda j: (j, 0)),
            pl.BlockSpec((1, 1, 448), lambda j: (j, 0, 0)),
        ],
        out_shape=[
            jax.ShapeDtypeStruct((50176, 96), F32),
            jax.ShapeDtypeStruct((112, 1, 448), jnp.int32),
        ],
        scratch_shapes=[pltpu.VMEM((1, 8192), F32)],
    )(z_enc, cb, bnp)


_REP = 8


def _rep_body(cb_ref, out_ref):
    out_ref[...] = cb_ref[...]


def _replicate(cb_pad):
    return pl.pallas_call(
        _rep_body,
        grid=(_REP,),
        in_specs=[pl.BlockSpec((8192, 128), lambda r: (0, 0))],
        out_specs=pl.BlockSpec((8192, 128), lambda r: (r, 0)),
        out_shape=jax.ShapeDtypeStruct((_REP * 8192, 128), F32),
    )(cb_pad)


def _sc_gather(codebook, idx_flat):
    info = plsc.get_sparse_core_info()
    nw = info.num_cores * info.num_subcores
    b_tot, d = idx_flat.shape[0], codebook.shape[1]
    b_per_w = b_tot // nw
    ch = 112
    n_ch = b_per_w // ch
    mesh = plsc.VectorSubcoreMesh(core_axis_name="c", subcore_axis_name="s")

    @functools.partial(
        pl.kernel, mesh=mesh,
        out_type=jax.ShapeDtypeStruct((b_tot, d), F32),
        scratch_types=[
            pltpu.VMEM((b_per_w,), jnp.int32),
            pltpu.VMEM((ch, d), F32),
            pltpu.VMEM((ch, d), F32),
            pltpu.SemaphoreType.DMA,
            pltpu.SemaphoreType.DMA,
        ],
    )
    def gk(cb_hbm, idx_hbm, out_hbm, idx_v, rows0, rows1, sem0, sem1):
        wid = lax.axis_index("s") * info.num_cores + lax.axis_index("c")
        base = wid * b_per_w
        pltpu.sync_copy(idx_hbm.at[pl.ds(base, b_per_w)], idx_v)
        off = (wid % _REP) * 8192
        for i in range(b_per_w // 16):
            sl = pl.ds(i * 16, 16)
            idx_v[sl] = idx_v[sl] + off
        bufs = (rows0, rows1)
        sems = (sem0, sem1)
        pend = None
        for c in range(n_ch):
            cur = pltpu.async_copy(
                cb_hbm.at[idx_v.at[pl.ds(c * ch, ch)]],
                bufs[c % 2], sems[c % 2])
            if pend is not None:
                pc, pcopy = pend
                pcopy.wait()
                pltpu.sync_copy(bufs[pc % 2],
                                out_hbm.at[pl.ds(base + pc * ch, ch)])
            pend = (c, cur)
        pc, pcopy = pend
        pcopy.wait()
        pltpu.sync_copy(bufs[pc % 2], out_hbm.at[pl.ds(base + pc * ch, ch)])

    return gk(codebook, idx_flat)


def _dec_body(zc_ref, qc_ref, zp_ref, qp_ref, zn_ref, qn_ref,
              w_ref, b2_ref, y_ref, st_ref):
    s = pl.program_id(0)
    jj = s % 14

    @pl.when(s == 0)
    def _():
        st_ref[...] = jnp.zeros_like(st_ref)

    zc = zc_ref[...]
    qc = qc_ref[:, :96]
    diff = qc - zc
    hq_c = zc + diff
    st_ref[2, :96] += jnp.sum(diff * diff, axis=0)

    hq_p = jnp.where(jj == 0, 0.0,
                     zp_ref[...] + (qp_ref[:, :96] - zp_ref[...]))
    hq_n = jnp.where(jj == 13, 0.0,
                     zn_ref[...] + (qn_ref[:, :96] - zn_ref[...]))
    hq = jnp.concatenate([hq_p, hq_c, hq_n], 0)

    hh = lax.broadcasted_iota(jnp.int32, (112, 226), 0)
    xx = lax.broadcasted_iota(jnp.int32, (112, 226), 1)
    dup_t = ((xx >= 1) & (xx <= 224) & (hh == (xx - 1) // 2)).astype(F32)

    w = w_ref[...]
    t_rows = []
    for hr in range(10):
        hw = _mm_hi(hq[112 * hr:112 * hr + 112, :], w)
        t_rows.append(_mm_exact_tn(hw, dup_t))

    b2 = b2_ref[0, :3]
    ssum = jnp.zeros((3,), F32)
    ssq = jnp.zeros((3,), F32)
    for i in range(16):
        acc = jnp.broadcast_to(b2[:, None], (3, 224))
        for ky in range(3):
            hloc = (i + ky - 1) // 2 + 1
            trow = t_rows[hloc]
            for kx in range(3):
                col = 3 * (3 * ky + kx)
                acc = acc + trow[col:col + 3, kx:kx + 224]
        y_ref[:, i, :] = acc
        ssum = ssum + jnp.sum(acc, axis=1)
        ssq = ssq + jnp.sum(acc * acc, axis=1)
    st_ref[0, :3] += ssum
    st_ref[1, :3] += ssq


def _decoder(z, q, w2r, b2):
    def hmap_prev(s):
        return (jnp.where(s % 14 == 0, 0, 8 * s - 1), 0)

    def hmap_next(s):
        return (jnp.where(s % 14 == 13, 0, 8 * s + 8), 0)

    return pl.pallas_call(
        _dec_body,
        grid=(56,),
        in_specs=[
            pl.BlockSpec((896, 96), lambda s: (s, 0)),
            pl.BlockSpec((896, 128), lambda s: (s, 0)),
            pl.BlockSpec((112, 96), hmap_prev),
            pl.BlockSpec((112, 128), hmap_prev),
            pl.BlockSpec((112, 96), hmap_next),
            pl.BlockSpec((112, 128), hmap_next),
            pl.BlockSpec((96, 27), lambda s: (0, 0)),
            pl.BlockSpec((1, 128), lambda s: (0, 0)),
        ],
        out_specs=[
            pl.BlockSpec((3, 16, 224), lambda s: (0, s, 0)),
            pl.BlockSpec((8, 128), lambda s: (0, 0)),
        ],
        out_shape=[
            jax.ShapeDtypeStruct((3, 896, 224), F32),
            jax.ShapeDtypeStruct((8, 128), F32),
        ],
    )(z, q, z, q, z, q, w2r, b2)


def _bn2_body(y_ref, bnp_ref, out_ref):
    g = bnp_ref[0, :3]
    bb = bnp_ref[1, :3]
    m = bnp_ref[2, :3]
    v = bnp_ref[3, :3]
    yv = y_ref[...]
    xh = (yv - m[:, None, None]) / jnp.sqrt(v[:, None, None] + EPS)
    out_ref[...] = jnp.tanh(xh * g[:, None, None] + bb[:, None, None])


def _bn2(y_pre, bnp2):
    return pl.pallas_call(
        _bn2_body,
        grid=(8,),
        in_specs=[
            pl.BlockSpec((3, 112, 224), lambda s: (0, s, 0)),
            pl.BlockSpec((8, 128), lambda s: (0, 0)),
        ],
        out_specs=pl.BlockSpec((3, 112, 224), lambda s: (0, s, 0)),
        out_shape=jax.ShapeDtypeStruct((3, 896, 224), F32),
    )(y_pre, bnp2)


def kernel(x, conv1_w, conv1_b, bn1_g, bn1_b, codebook,
           conv2_w, conv2_b, bn2_g, bn2_b):
    n1 = jnp.float32(4 * 224 * 224)

    x_pad = jnp.pad(x, ((0, 0), (0, 0), (1, 1), (1, 1)))
    w27 = jnp.transpose(conv1_w, (2, 3, 1, 0)).reshape(27, 96)
    z_enc, st1 = _encoder(x_pad, w27, conv1_b.reshape(1, 96))

    mean1 = st1[0, :96] / n1
    var1 = st1[1, :96] / n1 - mean1 * mean1
    bnp1 = jnp.zeros((8, 128), F32)
    bnp1 = bnp1.at[0, :96].set(bn1_g).at[1, :96].set(bn1_b)
    bnp1 = bnp1.at[2, :96].set(mean1).at[3, :96].set(var1)

    z, idx3 = _vq(z_enc, codebook, bnp1)
    idx_flat = idx3.reshape(50176)

    cb_pad = jnp.pad(codebook, ((0, 0), (0, 32)))
    q = _sc_gather(_replicate(cb_pad), idx_flat)

    w2r = jnp.transpose(conv2_w, (1, 2, 3, 0)).reshape(96, 27)
    y_pre, st2 = _decoder(z, q, w2r,
                          jnp.pad(conv2_b, (0, 125)).reshape(1, 128))
    commit_loss = jnp.float32(0.25) * (jnp.sum(st2[2, :96])
                                       / jnp.float32(50176 * 96))

    mean2 = st2[0, :3] / n1
    var2 = st2[1, :3] / n1 - mean2 * mean2
    bnp2 = jnp.zeros((8, 128), F32)
    bnp2 = bnp2.at[0, :3].set(bn2_g).at[1, :3].set(bn2_b)
    bnp2 = bnp2.at[2, :3].set(mean2).at[3, :3].set(var2)

    y_t = _bn2(y_pre, bnp2)
    y = jnp.transpose(y_t.reshape(3, 4, 224, 224), (1, 0, 2, 3))

    return y, idx_flat.reshape(4, 112, 112), commit_loss
